# baseline (speedup 1.0000x reference)
import functools

import jax
import jax.numpy as jnp
from jax import lax
from jax.experimental import pallas as pl
from jax.experimental.pallas import tpu as pltpu
from jax.experimental.pallas import tpu_sc as plsc


@functools.lru_cache(maxsize=None)
def _gather_kernel(V, D, T, B):
    info = plsc.get_sparse_core_info()
    NW = info.num_cores * info.num_subcores
    BS = B // NW
    assert B % NW == 0 and BS == 128 and T % 2 == 0 and D == 64
    TR = T // 8
    DB = D // 8

    mesh = plsc.VectorSubcoreMesh(core_axis_name="c", subcore_axis_name="s")

    @functools.partial(
        pl.kernel,
        mesh=mesh,
        out_type=jax.ShapeDtypeStruct((T * D * B,), jnp.float32),
        compiler_params=pltpu.CompilerParams(
            use_tc_tiling_on_sc=False, needs_layout_passes=False),
        scratch_types=[
            pltpu.VMEM((2, BS), jnp.int32),
            pltpu.VMEM((2, BS), jnp.int32),
            pltpu.VMEM((2, BS, 2 * D), jnp.float32),
            pltpu.VMEM((2, D * BS), jnp.float32),
            pltpu.SemaphoreType.DMA,
            pltpu.SemaphoreType.DMA,
            pltpu.SemaphoreType.DMA,
            pltpu.SemaphoreType.DMA,
            pltpu.SemaphoreType.DMA,
            pltpu.SemaphoreType.DMA,
        ],
    )
    def gkern(tab2, ids4, out1, idx_v, idxh_v, rows_v, rT_v,
              si0, si1, sg0, sg1, so0, so1):
        si = (si0, si1)
        sg = (sg0, sg1)
        so = (so0, so1)
        wid = lax.axis_index("s") * 2 + lax.axis_index("c")
        riota = lax.iota(jnp.int32, 16)

        def idx_start(t, b):
            pltpu.async_copy(
                ids4.at[t // 8, wid, t % 8], idx_v.at[b], si[b])

        def idx_wait(b):
            pltpu.make_async_copy(
                ids4.at[0, 0, 0], idx_v.at[b], si[b]).wait()

        def gather_start(b):
            pltpu.async_copy(tab2.at[idxh_v.at[b]], rows_v.at[b], sg[b])

        def gather_wait(b):
            pltpu.make_async_copy(
                tab2.at[idxh_v.at[b]], rows_v.at[b], sg[b]).wait()

        def out_drain(b):
            pltpu.make_async_copy(
                out1.at[pl.ds(0, D * BS)], rT_v.at[b], so[b]).wait()

        idx_start(0, 0)
        idx_start(1, 1)

        def body(i, carry):
            for b in range(2):
                @pl.when(i > 0)
                def _():
                    out_drain(b)
                idx_wait(b)
                for c in range(BS // 16):
                    v = idx_v.at[b][pl.ds(c * 16, 16)]
                    idxh_v.at[b][pl.ds(c * 16, 16)] = jnp.right_shift(v, 1)
                gather_start(b)
            for b in range(2):
                t = 2 * i + b
                gather_wait(b)
                src = rows_v.at[b]
                dst = rT_v.at[b]

                def tbody(bc, c2):
                    vi = idx_v.at[b][pl.ds(bc * 16, 16)]
                    par = jnp.bitwise_and(vi, 1) * 64
                    rowsel = riota + bc * 16
                    for d in range(D):
                        vals = plsc.load_gather(src, [rowsel, par + d])
                        dst[pl.ds(d * BS + bc * 16, 16)] = vals
                    return c2

                lax.fori_loop(0, BS // 16, tbody, 0)

                @pl.when(t + 2 < T)
                def _():
                    idx_start(t + 2, b)
                out_base = (t * DB * NW + wid) * (8 * BS)
                for db in range(DB):
                    pltpu.async_copy(
                        rT_v.at[b, pl.ds(db * 8 * BS, 8 * BS)],
                        out1.at[pl.ds(out_base + db * NW * 8 * BS, 8 * BS)],
                        so[b])
            return carry

        lax.fori_loop(0, T // 2, body, 0)
        out_drain(0)
        out_drain(1)

    return gkern


def kernel(input_ids, table):
    Bt, T = input_ids.shape
    V, D = table.shape
    tab2 = table.reshape(V // 2, 2 * D)
    ids4 = input_ids.reshape(Bt // 128, 128, T // 8, 8).transpose(2, 0, 3, 1)
    out1 = _gather_kernel(V, D, T, Bt)(tab2, ids4)
    o5 = out1.reshape(T, D // 8, Bt // 128, 8, 128)
    return o5.transpose(2, 4, 0, 1, 3).reshape(Bt, T, D)


# speedup vs baseline: 1.3943x; 1.3943x over previous
import functools

import jax
import jax.numpy as jnp
from jax import lax
from jax.experimental import pallas as pl
from jax.experimental.pallas import tpu as pltpu
from jax.experimental.pallas import tpu_sc as plsc


@functools.lru_cache(maxsize=None)
def _gather_kernel(V, D, T, B):
    info = plsc.get_sparse_core_info()
    NW = info.num_cores * info.num_subcores
    BS = B // NW
    assert B % NW == 0 and BS == 128 and T % 2 == 0 and D == 64
    TR = T // 8
    DB = D // 8

    mesh = plsc.VectorSubcoreMesh(core_axis_name="c", subcore_axis_name="s")

    @functools.partial(
        pl.kernel,
        mesh=mesh,
        out_type=jax.ShapeDtypeStruct((T * D * B,), jnp.float32),
        compiler_params=pltpu.CompilerParams(
            use_tc_tiling_on_sc=False, needs_layout_passes=False),
        scratch_types=[
            pltpu.VMEM((2, BS), jnp.int32),
            pltpu.VMEM((2, BS, D), jnp.float32),
            pltpu.VMEM((2, D * BS), jnp.float32),
            pltpu.SemaphoreType.DMA,
            pltpu.SemaphoreType.DMA,
            pltpu.SemaphoreType.DMA,
            pltpu.SemaphoreType.DMA,
            pltpu.SemaphoreType.DMA,
            pltpu.SemaphoreType.DMA,
        ],
    )
    def gkern(tab2, ids4, out1, idx_v, rows_v, rT_v,
              si0, si1, sg0, sg1, so0, so1):
        si = (si0, si1)
        sg = (sg0, sg1)
        so = (so0, so1)
        wid = lax.axis_index("s") * 2 + lax.axis_index("c")
        riota = lax.iota(jnp.int32, 16)

        def idx_start(t, b):
            pltpu.async_copy(
                ids4.at[t // 8, wid, t % 8], idx_v.at[b], si[b])

        def idx_wait(b):
            pltpu.make_async_copy(
                ids4.at[0, 0, 0], idx_v.at[b], si[b]).wait()

        def gather_start(b):
            pltpu.async_copy(tab2.at[idx_v.at[b]], rows_v.at[b], sg[b])

        def gather_wait(b):
            pltpu.make_async_copy(
                tab2.at[idx_v.at[b]], rows_v.at[b], sg[b]).wait()

        def out_drain(b):
            pltpu.make_async_copy(
                out1.at[pl.ds(0, D * BS)], rT_v.at[b], so[b]).wait()

        idx_start(0, 0)
        idx_start(1, 1)

        def body(i, carry):
            for b in range(2):
                @pl.when(i > 0)
                def _():
                    out_drain(b)
                idx_wait(b)
                gather_start(b)
            for b in range(2):
                t = 2 * i + b
                gather_wait(b)
                src = rows_v.at[b]
                dst = rT_v.at[b]

                def tbody(bc, c2):
                    rowsel = riota + bc * 16

                    @plsc.parallel_loop(0, D, unroll=8)
                    def _(d):
                        vals = plsc.load_gather(src, [rowsel, riota * 0 + d])
                        dst[pl.ds(d * BS + bc * 16, 16)] = vals
                    return c2

                lax.fori_loop(0, BS // 16, tbody, 0)

                @pl.when(t + 2 < T)
                def _():
                    idx_start(t + 2, b)
                out_base = (t * DB * NW + wid) * (8 * BS)
                for db in range(DB):
                    pltpu.async_copy(
                        rT_v.at[b, pl.ds(db * 8 * BS, 8 * BS)],
                        out1.at[pl.ds(out_base + db * NW * 8 * BS, 8 * BS)],
                        so[b])
            return carry

        lax.fori_loop(0, T // 2, body, 0)
        out_drain(0)
        out_drain(1)

    return gkern


def kernel(input_ids, table):
    Bt, T = input_ids.shape
    V, D = table.shape
    tab2 = table
    ids4 = input_ids.reshape(Bt // 128, 128, T // 8, 8).transpose(2, 0, 3, 1)
    out1 = _gather_kernel(V, D, T, Bt)(tab2, ids4)
    o5 = out1.reshape(T, D // 8, Bt // 128, 8, 128)
    return o5.transpose(2, 4, 0, 1, 3).reshape(Bt, T, D)


# speedup vs baseline: 1.4157x; 1.0154x over previous
import functools

import jax
import jax.numpy as jnp
from jax import lax
from jax.experimental import pallas as pl
from jax.experimental.pallas import tpu as pltpu
from jax.experimental.pallas import tpu_sc as plsc


@functools.lru_cache(maxsize=None)
def _gather_kernel(V, D, T, B):
    info = plsc.get_sparse_core_info()
    NW = info.num_cores * info.num_subcores
    BS = B // NW
    assert B % NW == 0 and BS == 128 and T % 2 == 0 and D == 64
    TR = T // 8
    DB = D // 8

    mesh = plsc.VectorSubcoreMesh(core_axis_name="c", subcore_axis_name="s")

    @functools.partial(
        pl.kernel,
        mesh=mesh,
        out_type=jax.ShapeDtypeStruct((T * D * B,), jnp.float32),
        compiler_params=pltpu.CompilerParams(
            use_tc_tiling_on_sc=False, needs_layout_passes=False,
            disable_bounds_checks=True),
        scratch_types=[
            pltpu.VMEM((2, BS), jnp.int32),
            pltpu.VMEM((2, BS, D), jnp.float32),
            pltpu.VMEM((2, D * BS), jnp.float32),
            pltpu.SemaphoreType.DMA,
            pltpu.SemaphoreType.DMA,
            pltpu.SemaphoreType.DMA,
            pltpu.SemaphoreType.DMA,
            pltpu.SemaphoreType.DMA,
            pltpu.SemaphoreType.DMA,
        ],
    )
    def gkern(tab2, ids4, out1, idx_v, rows_v, rT_v,
              si0, si1, sg0, sg1, so0, so1):
        si = (si0, si1)
        sg = (sg0, sg1)
        so = (so0, so1)
        wid = lax.axis_index("s") * 2 + lax.axis_index("c")
        riota = lax.iota(jnp.int32, 16)

        def idx_start(t, b):
            pltpu.async_copy(
                ids4.at[t // 8, wid, t % 8], idx_v.at[b], si[b])

        def idx_wait(b):
            pltpu.make_async_copy(
                ids4.at[0, 0, 0], idx_v.at[b], si[b]).wait()

        def gather_start(b):
            pltpu.async_copy(tab2.at[idx_v.at[b]], rows_v.at[b], sg[b])

        def gather_wait(b):
            pltpu.make_async_copy(
                tab2.at[idx_v.at[b]], rows_v.at[b], sg[b]).wait()

        def out_drain(b):
            pltpu.make_async_copy(
                out1.at[pl.ds(0, D * BS)], rT_v.at[b], so[b]).wait()

        idx_start(0, 0)
        idx_start(1, 1)

        def body(i, carry):
            for b in range(2):
                @pl.when(i > 0)
                def _():
                    out_drain(b)
                idx_wait(b)
                gather_start(b)
            for b in range(2):
                t = 2 * i + b
                gather_wait(b)
                src = rows_v.at[b]
                dst = rT_v.at[b]

                @plsc.parallel_loop(0, BS, unroll=8)
                def _(bb):
                    for c in range(D // 16):
                        vals = src.at[bb][pl.ds(c * 16, 16)]
                        plsc.store_scatter(
                            dst, [riota * BS + (c * 16 * BS + bb)], vals)

                @pl.when(t + 2 < T)
                def _():
                    idx_start(t + 2, b)
                out_base = (t * DB * NW + wid) * (8 * BS)
                for db in range(DB):
                    pltpu.async_copy(
                        rT_v.at[b, pl.ds(db * 8 * BS, 8 * BS)],
                        out1.at[pl.ds(out_base + db * NW * 8 * BS, 8 * BS)],
                        so[b])
            return carry

        lax.fori_loop(0, T // 2, body, 0)
        out_drain(0)
        out_drain(1)

    return gkern


def kernel(input_ids, table):
    Bt, T = input_ids.shape
    V, D = table.shape
    tab2 = table
    ids4 = input_ids.reshape(Bt // 128, 128, T // 8, 8).transpose(2, 0, 3, 1)
    out1 = _gather_kernel(V, D, T, Bt)(tab2, ids4)
    o5 = out1.reshape(T, D // 8, Bt // 128, 8, 128)
    return o5.transpose(2, 4, 0, 1, 3).reshape(Bt, T, D)


# speedup vs baseline: 1.5226x; 1.0755x over previous
import functools

import jax
import jax.numpy as jnp
from jax import lax
from jax.experimental import pallas as pl
from jax.experimental.pallas import tpu as pltpu
from jax.experimental.pallas import tpu_sc as plsc


@functools.lru_cache(maxsize=None)
def _gather_kernel(V, D, T, B):
    info = plsc.get_sparse_core_info()
    NW = info.num_cores * info.num_subcores
    BS = B // NW
    assert B % NW == 0 and BS == 128 and T % 2 == 0 and D == 64
    TR = T // 8
    DB = D // 8

    mesh = plsc.VectorSubcoreMesh(core_axis_name="c", subcore_axis_name="s")

    @functools.partial(
        pl.kernel,
        mesh=mesh,
        out_type=jax.ShapeDtypeStruct((T * D * B,), jnp.float32),
        compiler_params=pltpu.CompilerParams(
            use_tc_tiling_on_sc=False, needs_layout_passes=False,
            disable_bounds_checks=True),
        scratch_types=[
            pltpu.VMEM((4, BS), jnp.int32),
            pltpu.VMEM((4, BS, D), jnp.float32),
            pltpu.VMEM((4, D * BS), jnp.float32),
        ] + [pltpu.SemaphoreType.DMA] * 12,
    )
    def gkern(tab2, ids4, out1, idx_v, rows_v, rT_v, *sems):
        si = sems[0:4]
        sg = sems[4:8]
        so = sems[8:12]
        wid = lax.axis_index("s") * 2 + lax.axis_index("c")
        riota = lax.iota(jnp.int32, 16)

        def idx_start(t, b):
            pltpu.async_copy(
                ids4.at[t // 8, wid, t % 8], idx_v.at[b], si[b])

        def idx_wait(b):
            pltpu.make_async_copy(
                ids4.at[0, 0, 0], idx_v.at[b], si[b]).wait()

        def gather_start(b):
            pltpu.async_copy(tab2.at[idx_v.at[b]], rows_v.at[b], sg[b])

        def gather_wait(b):
            pltpu.make_async_copy(
                tab2.at[idx_v.at[b]], rows_v.at[b], sg[b]).wait()

        def out_drain(b):
            pltpu.make_async_copy(
                out1.at[pl.ds(0, D * BS)], rT_v.at[b], so[b]).wait()

        for b in range(4):
            idx_start(b, b)
        for b in range(2):
            idx_wait(b)
            gather_start(b)

        def body(i, carry):
            for b in range(4):
                t = 4 * i + b
                s2 = (b + 2) % 4

                @pl.when(i > 0)
                def _():
                    out_drain(b)

                @pl.when(t + 2 < T)
                def _():
                    idx_wait(s2)
                    gather_start(s2)
                gather_wait(b)
                src = rows_v.at[b]
                dst = rT_v.at[b]

                @plsc.parallel_loop(0, BS, unroll=8)
                def _(bb):
                    for c in range(D // 16):
                        vals = src.at[bb][pl.ds(c * 16, 16)]
                        plsc.store_scatter(
                            dst, [riota * BS + (c * 16 * BS + bb)], vals)

                @pl.when(t + 4 < T)
                def _():
                    idx_start(t + 4, b)
                out_base = (t * DB * NW + wid) * (8 * BS)
                for db in range(DB):
                    pltpu.async_copy(
                        rT_v.at[b, pl.ds(db * 8 * BS, 8 * BS)],
                        out1.at[pl.ds(out_base + db * NW * 8 * BS, 8 * BS)],
                        so[b])
            return carry

        lax.fori_loop(0, T // 4, body, 0)
        for b in range(4):
            out_drain(b)

    return gkern


def kernel(input_ids, table):
    Bt, T = input_ids.shape
    V, D = table.shape
    tab2 = table
    ids4 = input_ids.reshape(Bt // 128, 128, T // 8, 8).transpose(2, 0, 3, 1)
    out1 = _gather_kernel(V, D, T, Bt)(tab2, ids4)
    o5 = out1.reshape(T, D // 8, Bt // 128, 8, 128)
    return o5.transpose(2, 4, 0, 1, 3).reshape(Bt, T, D)


# speedup vs baseline: 1.6083x; 1.0563x over previous
import functools

import jax
import jax.numpy as jnp
from jax import lax
from jax.experimental import pallas as pl
from jax.experimental.pallas import tpu as pltpu
from jax.experimental.pallas import tpu_sc as plsc


@functools.lru_cache(maxsize=None)
def _make_gather(V, D, B):
    info = plsc.get_sparse_core_info()
    NC, NS = info.num_cores, info.num_subcores
    NW = NC * NS
    assert B % (NW * 128) == 0
    b_per_w = B // NW
    K = 4
    CH = K * 128
    G = b_per_w // CH
    assert b_per_w % CH == 0 and G % 2 == 0
    rows_w = b_per_w // 128

    mesh = plsc.VectorSubcoreMesh(core_axis_name="c", subcore_axis_name="s")

    @functools.partial(
        pl.kernel,
        mesh=mesh,
        out_type=jax.ShapeDtypeStruct((B, D), jnp.float32),
        compiler_params=pltpu.CompilerParams(use_tc_tiling_on_sc=False),
        scratch_types=[
            pltpu.VMEM((2, K, 128), jnp.int32),
            pltpu.VMEM((2, CH, D), jnp.float32),
            pltpu.SemaphoreType.DMA,
            pltpu.SemaphoreType.DMA,
            pltpu.SemaphoreType.DMA,
            pltpu.SemaphoreType.DMA,
            pltpu.SemaphoreType.DMA,
            pltpu.SemaphoreType.DMA,
        ],
    )
    def gather_kernel(table_hbm, idx_hbm, out_hbm, idx_v, rows_v,
                      si0, si1, sg0, sg1, so0, so1):
        si = (si0, si1)
        sg = (sg0, sg1)
        so = (so0, so1)
        wid = lax.axis_index("s") * NC + lax.axis_index("c")
        idx_row0 = wid * rows_w
        out_row0 = wid * b_per_w

        def idx_start(g, b):
            pltpu.async_copy(
                idx_hbm.at[pl.ds(idx_row0 + g * K, K)], idx_v.at[b], si[b])

        def idx_wait(b):
            pltpu.make_async_copy(
                idx_hbm.at[pl.ds(0, K)], idx_v.at[b], si[b]).wait()

        def out_drain(b):
            pltpu.make_async_copy(
                out_hbm.at[pl.ds(0, CH)], rows_v.at[b], so[b]).wait()

        idx_start(0, 0)
        idx_start(1, 1)

        def body(i, carry):
            for b in range(2):
                @pl.when(i > 0)
                def _():
                    out_drain(b)
                idx_wait(b)
                for j in range(K):
                    pltpu.async_copy(
                        table_hbm.at[idx_v.at[b, j]],
                        rows_v.at[b, pl.ds(j * 128, 128)],
                        sg[b])
            for b in range(2):
                g = 2 * i + b
                for j in range(K):
                    pltpu.make_async_copy(
                        table_hbm.at[idx_v.at[b, j]],
                        rows_v.at[b, pl.ds(j * 128, 128)],
                        sg[b]).wait()
                pltpu.async_copy(
                    rows_v.at[b], out_hbm.at[pl.ds(out_row0 + g * CH, CH)],
                    so[b])

                @pl.when(g + 2 < G)
                def _():
                    idx_start(g + 2, b)
            return carry

        lax.fori_loop(0, G // 2, body, 0)
        out_drain(0)
        out_drain(1)

    return gather_kernel


def kernel(input_ids, table):
    Bt, T = input_ids.shape
    B = Bt * T
    V, D = table.shape
    idx2d = input_ids.reshape(B // 128, 128).astype(jnp.int32)
    out = _make_gather(V, D, B)(table, idx2d)
    return out.reshape(Bt, T, D)
